# baseline (speedup 1.0000x reference)
import jax
import jax.numpy as jnp
from jax.experimental import pallas as pl
from jax.experimental.pallas import tpu as pltpu

_G = 4
_BM = 128


def _rowsum_support_kernel(*refs):
    adj_refs = refs[:_G]
    x_refs = refs[_G:2 * _G]
    w_ref = refs[2 * _G]
    r_refs = refs[2 * _G + 1:2 * _G + 1 + _G]
    t_refs = refs[2 * _G + 1 + _G:]
    w = w_ref[...]
    for g in range(_G):
        rs = jnp.sum(adj_refs[g][0], axis=1, keepdims=True) + 1.0
        rinv = jnp.power(rs, -0.5)
        rinv = jnp.where(jnp.isinf(rinv), 0.0, rinv)
        support = jnp.dot(x_refs[g][0], w, preferred_element_type=jnp.float32)
        r_refs[g][...] = rinv
        t_refs[g][...] = rinv * support


def _spmm_kernel(*refs):
    adj_refs = refs[:_G]
    t_ref = refs[_G]
    tblk_refs = refs[_G + 1:2 * _G + 1]
    r_refs = refs[2 * _G + 1:3 * _G + 1]
    b_ref = refs[3 * _G + 1]
    out_refs = refs[3 * _G + 2:]
    t = t_ref[...]
    bias = b_ref[...]
    for g in range(_G):
        acc = jnp.dot(adj_refs[g][0], t, preferred_element_type=jnp.float32)
        out_refs[g][...] = r_refs[g][...] * (acc + tblk_refs[g][...]) + bias


def kernel(input, adj, W, b):
    n, f_in = input.shape
    f_out = W.shape[1]
    rows = n // _G
    grid = (pl.cdiv(rows, _BM),)
    adj3 = adj.reshape(_G, rows, n)
    x3 = input.reshape(_G, rows, f_in)

    def gspec(g, bn):
        return pl.BlockSpec((1, _BM, bn), lambda m, g=g: (g, m, 0))

    def fspec(bn):
        return pl.BlockSpec((_BM, bn), lambda m: (m, 0))

    outs = pl.pallas_call(
        _rowsum_support_kernel,
        grid=grid,
        in_specs=(
            [gspec(g, n) for g in range(_G)]
            + [gspec(g, f_in) for g in range(_G)]
            + [pl.BlockSpec((f_in, f_out), lambda m: (0, 0))]
        ),
        out_specs=[fspec(1)] * _G + [fspec(f_out)] * _G,
        out_shape=(
            [jax.ShapeDtypeStruct((rows, 1), jnp.float32) for _ in range(_G)]
            + [jax.ShapeDtypeStruct((rows, f_out), jnp.float32) for _ in range(_G)]
        ),
        compiler_params=pltpu.CompilerParams(
            dimension_semantics=("arbitrary",),
        ),
    )(*([adj3] * _G), *([x3] * _G), W)
    r_parts, t_parts = outs[:_G], outs[_G:]
    t2 = jnp.concatenate(t_parts, axis=0)

    b2 = b.reshape(1, f_out)
    out_parts = pl.pallas_call(
        _spmm_kernel,
        grid=grid,
        in_specs=(
            [gspec(g, n) for g in range(_G)]
            + [pl.BlockSpec((n, f_out), lambda m: (0, 0))]
            + [fspec(f_out)] * _G
            + [fspec(1)] * _G
            + [pl.BlockSpec((1, f_out), lambda m: (0, 0))]
        ),
        out_specs=[fspec(f_out)] * _G,
        out_shape=[jax.ShapeDtypeStruct((rows, f_out), jnp.float32)
                   for _ in range(_G)],
        compiler_params=pltpu.CompilerParams(
            dimension_semantics=("arbitrary",),
        ),
    )(*([adj3] * _G), t2, *t_parts, *r_parts, b2)
    return jnp.concatenate(out_parts, axis=0)


# speedup vs baseline: 2.3755x; 2.3755x over previous
import jax
import jax.numpy as jnp
from jax.experimental import pallas as pl
from jax.experimental.pallas import tpu as pltpu

_BM = 200


def _rowsum_support_kernel(adj_ref, x_ref, w_ref, r_ref, t_ref):
    rs = jnp.sum(adj_ref[...], axis=1, keepdims=True) + 1.0
    rinv = jnp.power(rs, -0.5)
    rinv = jnp.where(jnp.isinf(rinv), 0.0, rinv)
    support = jnp.dot(x_ref[...], w_ref[...], preferred_element_type=jnp.float32)
    r_ref[...] = rinv
    t_ref[...] = rinv * support


def _spmm_kernel(adj_ref, t_ref, t_blk_ref, r_ref, b_ref, out_ref):
    acc = jnp.dot(adj_ref[...], t_ref[...], preferred_element_type=jnp.float32)
    out_ref[...] = r_ref[...] * (acc + t_blk_ref[...]) + b_ref[...]


def kernel(input, adj, W, b):
    n, f_in = input.shape
    f_out = W.shape[1]
    grid = (n // _BM,)

    r, t = pl.pallas_call(
        _rowsum_support_kernel,
        grid=grid,
        in_specs=[
            pl.BlockSpec((_BM, n), lambda m: (m, 0)),
            pl.BlockSpec((_BM, f_in), lambda m: (m, 0)),
            pl.BlockSpec((f_in, f_out), lambda m: (0, 0)),
        ],
        out_specs=[
            pl.BlockSpec((_BM, 1), lambda m: (m, 0)),
            pl.BlockSpec((_BM, f_out), lambda m: (m, 0)),
        ],
        out_shape=[
            jax.ShapeDtypeStruct((n, 1), jnp.float32),
            jax.ShapeDtypeStruct((n, f_out), jnp.float32),
        ],
        compiler_params=pltpu.CompilerParams(
            dimension_semantics=("arbitrary",),
        ),
    )(adj, input, W)

    b2 = b.reshape(1, f_out)
    out = pl.pallas_call(
        _spmm_kernel,
        grid=grid,
        in_specs=[
            pl.BlockSpec((_BM, n), lambda m: (m, 0)),
            pl.BlockSpec((n, f_out), lambda m: (0, 0)),
            pl.BlockSpec((_BM, f_out), lambda m: (m, 0)),
            pl.BlockSpec((_BM, 1), lambda m: (m, 0)),
            pl.BlockSpec((1, f_out), lambda m: (0, 0)),
        ],
        out_specs=pl.BlockSpec((_BM, f_out), lambda m: (m, 0)),
        out_shape=jax.ShapeDtypeStruct((n, f_out), jnp.float32),
        compiler_params=pltpu.CompilerParams(
            dimension_semantics=("arbitrary",),
        ),
    )(adj, t, t, r, b2)
    return out
